# Initial kernel scaffold; baseline (speedup 1.0000x reference)
import jax
import jax.numpy as jnp
from jax.experimental import pallas as pl


def kernel(output, nms_thresh):
    raise NotImplementedError("write your pallas kernel here")


# speedup vs baseline: 1.2320x; 1.2320x over previous
import functools

import jax
import jax.numpy as jnp
from jax import lax
from jax.experimental import pallas as pl
from jax.experimental.pallas import tpu as pltpu
from jax.experimental.pallas import tpu_sc as plsc

_ANCHOR_MASK = [3, 4, 5]
_ANCHORS = [10.0, 13.0, 16.0, 30.0, 33.0, 23.0, 30.0, 61.0, 62.0, 45.0,
            59.0, 119.0, 116.0, 90.0, 156.0, 198.0, 373.0, 326.0]
_STRIDE = 32
_NCLS = 80
_CH = 5 + _NCLS

_B, _NA, _H, _W = 8, 3, 64, 64
_HW = _H * _W
_NSLAB = _B * _NA
_NPOS = _NSLAB * _HW

_NWORKERS = 32
_PER_W = _NPOS // _NWORKERS
_CHUNK = 512
_NSUB = _PER_W // _CHUNK
_GROUPS = _CHUNK // 16

_AW = [_ANCHORS[m * 2] / _STRIDE / _W for m in _ANCHOR_MASK]
_AH = [_ANCHORS[m * 2 + 1] / _STRIDE / _H for m in _ANCHOR_MASK]


def _work(wid, x_hbm, thr_hbm, out_hbm, in_v0, in_v1, out_v, thr_v, sem0, sem1):
    pltpu.sync_copy(thr_hbm, thr_v)
    thr = thr_v[...]

    lane_i = lax.iota(jnp.int32, 16)
    lane_f = lane_i.astype(jnp.float32)
    seven_i = lane_i * 7

    def src(j):
        g = wid * _PER_W + j * _CHUNK
        s = g // _HW
        p0 = g % _HW
        return g, s, p0

    def start(j, buf, sem):
        _, s, p0 = src(j)
        pltpu.async_copy(x_hbm.at[s, :, pl.ds(p0, _CHUNK)], buf, sem)

    def wait(j, buf, sem):
        _, s, p0 = src(j)
        pltpu.make_async_copy(x_hbm.at[s, :, pl.ds(p0, _CHUNK)], buf, sem).wait()

    def compute(j, in_v):
        g, s, p0 = src(j)
        a = s % _NA
        aw = jnp.where(a == 0, _AW[0], jnp.where(a == 1, _AW[1], _AW[2]))
        ah = jnp.where(a == 0, _AH[0], jnp.where(a == 1, _AH[1], _AH[2]))

        def group(i, carry2):
            q = i * 16
            p = p0 + q
            gx0 = (p % _W).astype(jnp.float32)
            gy0 = (p // _W).astype(jnp.float32)

            x0 = in_v[0, pl.ds(q, 16)]
            x1 = in_v[1, pl.ds(q, 16)]
            x2 = in_v[2, pl.ds(q, 16)]
            x3 = in_v[3, pl.ds(q, 16)]
            x4 = in_v[4, pl.ds(q, 16)]

            one = jnp.float32(1.0)
            sig0 = one / (one + jnp.exp(-x0))
            sig1 = one / (one + jnp.exp(-x1))
            det = one / (one + jnp.exp(-x4))

            xs = (sig0 + gx0 + lane_f) * jnp.float32(1.0 / _W)
            ys = (sig1 + gy0) * jnp.float32(1.0 / _H)
            ws = jnp.exp(x2) * aw
            hs = jnp.exp(x3) * ah

            parts = []
            for h in range(4):
                lo = h * (_NCLS // 4)
                m = in_v[5 + lo, pl.ds(q, 16)]
                cid_v = jnp.full((16,), jnp.float32(lo))
                s0 = jnp.exp(m)
                s1 = jnp.zeros((16,), jnp.float32)
                for k in range(1, _NCLS // 4):
                    c = lo + k
                    l = in_v[5 + c, pl.ds(q, 16)]
                    e = jnp.exp(l)
                    if k % 2 == 0:
                        s0 = s0 + e
                    else:
                        s1 = s1 + e
                    upd = l > m
                    m = jnp.maximum(l, m)
                    cid_v = jnp.where(upd, jnp.float32(c), cid_v)
                parts.append((m, cid_v, s0 + s1))

            def merge(a, b):
                ma, ca, sa = a
                mb, cb, sb = b
                u = mb > ma
                return (jnp.where(u, mb, ma), jnp.where(u, cb, ca), sa + sb)

            m, cid_v, ssum = merge(merge(parts[0], parts[1]),
                                   merge(parts[2], parts[3]))
            cconf = jnp.exp(m) / ssum

            keep = jnp.where(det > thr, one, jnp.float32(0.0))

            base = seven_i + q * 7
            plsc.store_scatter(out_v, [base], xs * keep)
            plsc.store_scatter(out_v, [base + 1], ys * keep)
            plsc.store_scatter(out_v, [base + 2], ws * keep)
            plsc.store_scatter(out_v, [base + 3], hs * keep)
            plsc.store_scatter(out_v, [base + 4], det * keep)
            plsc.store_scatter(out_v, [base + 5], cconf * keep)
            plsc.store_scatter(out_v, [base + 6], cid_v * keep)
            return carry2

        lax.fori_loop(0, _GROUPS, group, 0)
        pltpu.sync_copy(out_v, out_hbm.at[pl.ds(g * 7, _CHUNK * 7)])

    start(0, in_v0, sem0)
    start(1, in_v1, sem1)

    def outer(jj, carry):
        j0 = jj * 2
        wait(j0, in_v0, sem0)
        compute(j0, in_v0)

        @pl.when(j0 + 2 < _NSUB)
        def _():
            start(j0 + 2, in_v0, sem0)

        j1 = jj * 2 + 1
        wait(j1, in_v1, sem1)
        compute(j1, in_v1)

        @pl.when(j1 + 2 < _NSUB)
        def _():
            start(j1 + 2, in_v1, sem1)

        return carry

    lax.fori_loop(0, _NSUB // 2, outer, 0)


def _body(x_hbm, thr_hbm, out_hbm, in_v0, in_v1, out_v, thr_v, sem0, sem1):
    wid = lax.axis_index("s") * 2 + lax.axis_index("c")
    _work(wid, x_hbm, thr_hbm, out_hbm, in_v0, in_v1, out_v, thr_v, sem0, sem1)


@jax.jit
def _yolo_sc(x, thr):
    mesh = plsc.VectorSubcoreMesh(core_axis_name="c", subcore_axis_name="s")
    return pl.kernel(
        _body,
        out_type=jax.ShapeDtypeStruct((_NPOS * 7,), jnp.float32),
        mesh=mesh,
        scratch_types=[
            pltpu.VMEM((_CH, _CHUNK), jnp.float32),
            pltpu.VMEM((_CH, _CHUNK), jnp.float32),
            pltpu.VMEM((_CHUNK * 7,), jnp.float32),
            pltpu.VMEM((16,), jnp.float32),
            pltpu.SemaphoreType.DMA,
            pltpu.SemaphoreType.DMA,
        ],
        compiler_params=pltpu.CompilerParams(needs_layout_passes=False),
    )(x, thr)


def kernel(output, nms_thresh):
    x = output.reshape(_NSLAB, _CH, _HW)
    thr = jnp.full((16,), nms_thresh, dtype=jnp.float32)
    flat = _yolo_sc(x, thr)
    return flat.reshape(_B, _NA * _HW, 7)


# speedup vs baseline: 1.3231x; 1.0740x over previous
import functools

import jax
import jax.numpy as jnp
from jax import lax
from jax.experimental import pallas as pl
from jax.experimental.pallas import tpu as pltpu
from jax.experimental.pallas import tpu_sc as plsc

_ANCHOR_MASK = [3, 4, 5]
_ANCHORS = [10.0, 13.0, 16.0, 30.0, 33.0, 23.0, 30.0, 61.0, 62.0, 45.0,
            59.0, 119.0, 116.0, 90.0, 156.0, 198.0, 373.0, 326.0]
_STRIDE = 32
_NCLS = 80
_CH = 5 + _NCLS

_B, _NA, _H, _W = 8, 3, 64, 64
_HW = _H * _W
_NSLAB = _B * _NA
_NPOS = _NSLAB * _HW

_NWORKERS = 32
_PER_W = _NPOS // _NWORKERS
_CHUNK = 256
_NSUB = _PER_W // _CHUNK
_GROUPS = _CHUNK // 16

_AW = [_ANCHORS[m * 2] / _STRIDE / _W for m in _ANCHOR_MASK]
_AH = [_ANCHORS[m * 2 + 1] / _STRIDE / _H for m in _ANCHOR_MASK]


def _work(wid, x_hbm, thr_hbm, out_hbm, in_v0, in_v1, out_v, thr_v, sem0, sem1):
    pltpu.sync_copy(thr_hbm, thr_v)
    thr = thr_v[...]

    lane_i = lax.iota(jnp.int32, 16)
    lane_f = lane_i.astype(jnp.float32)
    seven_i = lane_i * 7

    def src(j):
        g = wid * _PER_W + j * _CHUNK
        s = g // _HW
        p0 = g % _HW
        return g, s, p0

    def window(j):
        _, s, p0 = src(j)
        b = s // _NA
        a = s % _NA
        h0 = pl.multiple_of(p0 // _W, _CHUNK // _W)
        return x_hbm.at[b, pl.ds(a * _CH, _CH), pl.ds(h0, _CHUNK // _W), :]

    def start(j, buf, sem):
        pltpu.async_copy(window(j), buf, sem)

    def wait(j, buf, sem):
        pltpu.make_async_copy(window(j), buf, sem).wait()

    def compute(j, in_v):
        g, s, p0 = src(j)
        a = s % _NA
        aw = jnp.where(a == 0, _AW[0], jnp.where(a == 1, _AW[1], _AW[2]))
        ah = jnp.where(a == 0, _AH[0], jnp.where(a == 1, _AH[1], _AH[2]))

        def group(i, carry2):
            q = i * 16
            p = p0 + q
            gx0 = (p % _W).astype(jnp.float32)
            gy0 = (p // _W).astype(jnp.float32)
            r = q // _W
            cc = q % _W

            def ld(c):
                return in_v[c, r, pl.ds(cc, 16)]

            x0 = ld(0)
            x1 = ld(1)
            x2 = ld(2)
            x3 = ld(3)
            x4 = ld(4)

            one = jnp.float32(1.0)
            sig0 = one / (one + jnp.exp(-x0))
            sig1 = one / (one + jnp.exp(-x1))
            det = one / (one + jnp.exp(-x4))

            xs = (sig0 + gx0 + lane_f) * jnp.float32(1.0 / _W)
            ys = (sig1 + gy0) * jnp.float32(1.0 / _H)
            ws = jnp.exp(x2) * aw
            hs = jnp.exp(x3) * ah

            parts = []
            for h in range(4):
                lo = h * (_NCLS // 4)
                m = ld(5 + lo)
                cid_v = jnp.full((16,), jnp.float32(lo))
                s0 = jnp.exp(m)
                s1 = jnp.zeros((16,), jnp.float32)
                for k in range(1, _NCLS // 4):
                    c = lo + k
                    l = ld(5 + c)
                    e = jnp.exp(l)
                    if k % 2 == 0:
                        s0 = s0 + e
                    else:
                        s1 = s1 + e
                    upd = l > m
                    m = jnp.maximum(l, m)
                    cid_v = jnp.where(upd, jnp.float32(c), cid_v)
                parts.append((m, cid_v, s0 + s1))

            def merge(a, b):
                ma, ca, sa = a
                mb, cb, sb = b
                u = mb > ma
                return (jnp.where(u, mb, ma), jnp.where(u, cb, ca), sa + sb)

            m, cid_v, ssum = merge(merge(parts[0], parts[1]),
                                   merge(parts[2], parts[3]))
            cconf = jnp.exp(m) / ssum

            keep = jnp.where(det > thr, one, jnp.float32(0.0))

            base = seven_i + q * 7
            plsc.store_scatter(out_v, [base], xs * keep)
            plsc.store_scatter(out_v, [base + 1], ys * keep)
            plsc.store_scatter(out_v, [base + 2], ws * keep)
            plsc.store_scatter(out_v, [base + 3], hs * keep)
            plsc.store_scatter(out_v, [base + 4], det * keep)
            plsc.store_scatter(out_v, [base + 5], cconf * keep)
            plsc.store_scatter(out_v, [base + 6], cid_v * keep)
            return carry2

        lax.fori_loop(0, _GROUPS, group, 0)
        pltpu.sync_copy(out_v, out_hbm.at[pl.ds(g * 7, _CHUNK * 7)])

    start(0, in_v0, sem0)
    start(1, in_v1, sem1)

    def outer(jj, carry):
        j0 = jj * 2
        wait(j0, in_v0, sem0)
        compute(j0, in_v0)

        @pl.when(j0 + 2 < _NSUB)
        def _():
            start(j0 + 2, in_v0, sem0)

        j1 = jj * 2 + 1
        wait(j1, in_v1, sem1)
        compute(j1, in_v1)

        @pl.when(j1 + 2 < _NSUB)
        def _():
            start(j1 + 2, in_v1, sem1)

        return carry

    lax.fori_loop(0, _NSUB // 2, outer, 0)


def _body(x_hbm, thr_hbm, out_hbm, in_v0, in_v1, out_v, thr_v, sem0, sem1):
    wid = lax.axis_index("s") * 2 + lax.axis_index("c")
    _work(wid, x_hbm, thr_hbm, out_hbm, in_v0, in_v1, out_v, thr_v, sem0, sem1)


@jax.jit
def _yolo_sc(x, thr):
    mesh = plsc.VectorSubcoreMesh(core_axis_name="c", subcore_axis_name="s")
    return pl.kernel(
        _body,
        out_type=jax.ShapeDtypeStruct((_NPOS * 7,), jnp.float32),
        mesh=mesh,
        scratch_types=[
            pltpu.VMEM((_CH, _CHUNK // _W, _W), jnp.float32),
            pltpu.VMEM((_CH, _CHUNK // _W, _W), jnp.float32),
            pltpu.VMEM((_CHUNK * 7,), jnp.float32),
            pltpu.VMEM((16,), jnp.float32),
            pltpu.SemaphoreType.DMA,
            pltpu.SemaphoreType.DMA,
        ],
        compiler_params=pltpu.CompilerParams(needs_layout_passes=False),
    )(x, thr)


def kernel(output, nms_thresh):
    thr = jnp.full((16,), nms_thresh, dtype=jnp.float32)
    flat = _yolo_sc(output, thr)
    return flat.reshape(_B, _NA * _HW, 7)


# speedup vs baseline: 2.3243x; 1.7566x over previous
import functools

import jax
import jax.numpy as jnp
from jax import lax
from jax.experimental import pallas as pl
from jax.experimental.pallas import tpu as pltpu
from jax.experimental.pallas import tpu_sc as plsc

_ANCHOR_MASK = [3, 4, 5]
_ANCHORS = [10.0, 13.0, 16.0, 30.0, 33.0, 23.0, 30.0, 61.0, 62.0, 45.0,
            59.0, 119.0, 116.0, 90.0, 156.0, 198.0, 373.0, 326.0]
_STRIDE = 32
_NCLS = 80
_CH = 5 + _NCLS

_B, _NA, _H, _W = 8, 3, 64, 64
_HW = _H * _W
_NSLAB = _B * _NA
_NPOS = _NSLAB * _HW

_NWORKERS = 32
_PER_W = _NPOS // _NWORKERS
_CHUNK = 256
_NSUB = _PER_W // _CHUNK
_GROUPS = _CHUNK // 16

_AW = [_ANCHORS[m * 2] / _STRIDE / _W for m in _ANCHOR_MASK]
_AH = [_ANCHORS[m * 2 + 1] / _STRIDE / _H for m in _ANCHOR_MASK]


def _work(wid, x_hbm, thr_hbm, out_hbm, in_v0, in_v1, out_v, thr_v, sem0, sem1):
    pltpu.sync_copy(thr_hbm, thr_v)
    thr = thr_v[...]

    lane_i = lax.iota(jnp.int32, 16)
    lane_f = lane_i.astype(jnp.float32)
    seven_i = lane_i * 7

    def src(j):
        g = wid * _PER_W + j * _CHUNK
        s = g // _HW
        p0 = g % _HW
        return g, s, p0

    def window(j):
        _, s, p0 = src(j)
        b = s // _NA
        a = s % _NA
        h0 = pl.multiple_of(p0 // _W, _CHUNK // _W)
        return x_hbm.at[b, pl.ds(a * _CH, _CH), pl.ds(h0, _CHUNK // _W), :]

    def start(j, buf, sem):
        pltpu.async_copy(window(j), buf, sem)

    def wait(j, buf, sem):
        pltpu.make_async_copy(window(j), buf, sem).wait()

    def compute(j, in_v):
        g, s, p0 = src(j)
        a = s % _NA
        aw = jnp.where(a == 0, _AW[0], jnp.where(a == 1, _AW[1], _AW[2]))
        ah = jnp.where(a == 0, _AH[0], jnp.where(a == 1, _AH[1], _AH[2]))

        def group(i, carry2):
            q = i * 16
            p = p0 + q
            gx0 = (p % _W).astype(jnp.float32)
            gy0 = (p // _W).astype(jnp.float32)
            r = q // _W
            cc = q % _W

            def ld(c):
                return in_v[c, r, pl.ds(cc, 16)]

            x0 = ld(0)
            x1 = ld(1)
            x2 = ld(2)
            x3 = ld(3)
            x4 = ld(4)

            one = jnp.float32(1.0)
            sig0 = one / (one + jnp.exp(-x0))
            sig1 = one / (one + jnp.exp(-x1))
            det = one / (one + jnp.exp(-x4))

            xs = (sig0 + gx0 + lane_f) * jnp.float32(1.0 / _W)
            ys = (sig1 + gy0) * jnp.float32(1.0 / _H)
            ws = jnp.exp(x2) * aw
            hs = jnp.exp(x3) * ah

            parts = []
            for h in range(4):
                lo = h * (_NCLS // 4)
                m = ld(5 + lo)
                cid_v = jnp.full((16,), jnp.float32(lo))
                s0 = jnp.exp(m)
                s1 = jnp.zeros((16,), jnp.float32)
                for k in range(1, _NCLS // 4):
                    c = lo + k
                    l = ld(5 + c)
                    e = jnp.exp(l)
                    if k % 2 == 0:
                        s0 = s0 + e
                    else:
                        s1 = s1 + e
                    upd = l > m
                    m = jnp.maximum(l, m)
                    cid_v = jnp.where(upd, jnp.float32(c), cid_v)
                parts.append((m, cid_v, s0 + s1))

            def merge(a, b):
                ma, ca, sa = a
                mb, cb, sb = b
                u = mb > ma
                return (jnp.where(u, mb, ma), jnp.where(u, cb, ca), sa + sb)

            m, cid_v, ssum = merge(merge(parts[0], parts[1]),
                                   merge(parts[2], parts[3]))
            cconf = jnp.exp(m) / ssum

            keep = jnp.where(det > thr, one, jnp.float32(0.0))

            t = q // 128
            l0 = q % 128
            vals = (xs, ys, ws, hs, det, cconf, cid_v)
            for f in range(7):
                out_v[f, t, pl.ds(l0, 16)] = vals[f] * keep
            return carry2

        lax.fori_loop(0, _GROUPS, group, 0)
        b = s // _NA
        k0 = a * (_HW // 128) + pl.multiple_of(p0 // 128, _CHUNK // 128)
        pltpu.sync_copy(out_v, out_hbm.at[:, pl.ds(k0, _CHUNK // 128), b, :])

    start(0, in_v0, sem0)
    start(1, in_v1, sem1)

    def outer(jj, carry):
        j0 = jj * 2
        wait(j0, in_v0, sem0)
        compute(j0, in_v0)

        @pl.when(j0 + 2 < _NSUB)
        def _():
            start(j0 + 2, in_v0, sem0)

        j1 = jj * 2 + 1
        wait(j1, in_v1, sem1)
        compute(j1, in_v1)

        @pl.when(j1 + 2 < _NSUB)
        def _():
            start(j1 + 2, in_v1, sem1)

        return carry

    lax.fori_loop(0, _NSUB // 2, outer, 0)


def _body(x_hbm, thr_hbm, out_hbm, in_v0, in_v1, out_v, thr_v, sem0, sem1):
    wid = lax.axis_index("s") * 2 + lax.axis_index("c")
    _work(wid, x_hbm, thr_hbm, out_hbm, in_v0, in_v1, out_v, thr_v, sem0, sem1)


@jax.jit
def _yolo_sc(x, thr):
    mesh = plsc.VectorSubcoreMesh(core_axis_name="c", subcore_axis_name="s")
    return pl.kernel(
        _body,
        out_type=jax.ShapeDtypeStruct((7, _NA * _HW // 128, _B, 128), jnp.float32),
        mesh=mesh,
        scratch_types=[
            pltpu.VMEM((_CH, _CHUNK // _W, _W), jnp.float32),
            pltpu.VMEM((_CH, _CHUNK // _W, _W), jnp.float32),
            pltpu.VMEM((7, _CHUNK // 128, 128), jnp.float32),
            pltpu.VMEM((16,), jnp.float32),
            pltpu.SemaphoreType.DMA,
            pltpu.SemaphoreType.DMA,
        ],
        compiler_params=pltpu.CompilerParams(needs_layout_passes=False),
    )(x, thr)


def kernel(output, nms_thresh):
    thr = jnp.full((16,), nms_thresh, dtype=jnp.float32)
    out4 = _yolo_sc(output, thr)
    return out4.transpose(2, 1, 3, 0).reshape(_B, _NA * _HW, 7)
